# baseline (speedup 1.0000x reference)
import functools

import jax
import jax.numpy as jnp
from jax import lax
from jax.experimental import pallas as pl
from jax.experimental.pallas import tpu as pltpu
from jax.experimental.pallas import tpu_sc as plsc

_N = 10000
_E = 320000
_D = 128
_G = 128
_NC = 2
_NS = 16
_NW = _NC * _NS
_K = 80
_EPT = 10000
_NCH = _EPT // _K
_NB = 3
_ZR = 632
_ZL = _N - 15 * _ZR


def _seg_sum_sc(h, src, dst):
    mesh = plsc.VectorSubcoreMesh(
        core_axis_name="c", subcore_axis_name="s",
        num_cores=_NC, num_subcores=_NS)

    @functools.partial(
        pl.kernel, mesh=mesh,
        out_type=jax.ShapeDtypeStruct((_NC, _N, _D), jnp.float32),
        scratch_types=[
            pltpu.VMEM((_EPT,), jnp.int32),
            pltpu.VMEM((_EPT,), jnp.int32),
            pltpu.VMEM((_K,), jnp.int32),
            [pltpu.VMEM((_K, _D), jnp.float32) for _ in range(_NB)],
            pltpu.VMEM_SHARED((_N, _D), jnp.float32),
            pltpu.SemaphoreType.DMA,
            pltpu.SemaphoreType.DMA,
        ],
    )
    def k(h_hbm, src_hbm, dst_hbm, out_hbm, src_v, dst_v, ibuf, rows,
          acc_sh, sem_g, sem_s):
        cid = lax.axis_index("c")
        sid = lax.axis_index("s")
        wid = sid * _NC + cid
        base = pl.multiple_of(sid * _ZR, 8)

        def zrow(i, _):
            def zcol(j, _):
                rows[0][i, pl.ds(j * 16, 16)] = jnp.zeros((16,), jnp.float32)
                return 0
            return lax.fori_loop(0, _D // 16, zcol, 0)
        lax.fori_loop(0, _K, zrow, 0)

        def zacc(nz, r):
            for i in range(nz):
                pltpu.async_copy(
                    rows[0], acc_sh.at[pl.ds(base + i * _K, _K)], sem_s)
            pltpu.async_copy(rows[0].at[pl.ds(0, r)],
                             acc_sh.at[pl.ds(base + nz * _K, r)], sem_s)
            for i in range(nz):
                pltpu.make_async_copy(
                    rows[0], acc_sh.at[pl.ds(0, _K)], sem_s).wait()
            pltpu.make_async_copy(rows[0].at[pl.ds(0, r)],
                                  acc_sh.at[pl.ds(0, r)], sem_s).wait()

        @pl.when(sid < _NS - 1)
        def _():
            zacc(_ZR // _K, _ZR - (_ZR // _K) * _K)

        @pl.when(sid == _NS - 1)
        def _():
            zacc(_ZL // _K, _ZL - (_ZL // _K) * _K)

        pltpu.sync_copy(src_hbm.at[wid], src_v)
        pltpu.sync_copy(dst_hbm.at[wid], dst_v)
        plsc.subcore_barrier()

        def chunk(j):
            return pl.ds(pl.multiple_of(jnp.minimum(j, _NCH - 1) * _K, _K),
                         _K)

        def g_issue(j, b):
            pltpu.async_copy(h_hbm.at[src_v.at[chunk(j)]], rows[b], sem_g)

        def g_wait(b):
            pltpu.make_async_copy(
                h_hbm.at[src_v.at[pl.ds(0, _K)]], rows[b], sem_g).wait()

        def bounce(j):
            off = pl.multiple_of(j * _K, _K)
            for t in range(_K // 16):
                ibuf[pl.ds(t * 16, 16)] = dst_v[pl.ds(off + t * 16, 16)]

        def s_issue(b):
            pltpu.async_copy(rows[b], acc_sh.at[ibuf], sem_s, add=True)

        def s_wait(b):
            pltpu.make_async_copy(rows[b], acc_sh.at[ibuf], sem_s).wait()

        def step(j, b):
            bp = (b - 1) % _NB
            g_wait(b)
            s_wait(bp)
            g_issue(j + 2, bp)
            bounce(j)
            s_issue(b)

        g_issue(0, 0)
        g_issue(1, 1)
        g_wait(0)
        bounce(0)
        s_issue(0)
        g_issue(2, 2)

        def body(i, _):
            for u in range(3):
                j = i * 3 + u + 1
                step(j, (u + 1) % _NB)
            return 0
        lax.fori_loop(0, (_NCH - 1) // 3, body, 0)

        for j in range(1 + 3 * ((_NCH - 1) // 3), _NCH):
            step(j, j % _NB)
        s_wait((_NCH - 1) % _NB)
        g_wait(0)
        g_wait(1)
        plsc.subcore_barrier()

        @pl.when(sid < _NS - 1)
        def _():
            pltpu.sync_copy(acc_sh.at[pl.ds(base, _ZR)],
                            out_hbm.at[cid, pl.ds(base, _ZR)])

        @pl.when(sid == _NS - 1)
        def _():
            pltpu.sync_copy(acc_sh.at[pl.ds(base, _ZL)],
                            out_hbm.at[cid, pl.ds(base, _ZL)])

    return k(h, src, dst)


def _layer_tc(h, agg2, w1, b1, w2, b2, g, be):
    def body(h_ref, a_ref, w1_ref, b1_ref, w2_ref, b2_ref, g_ref, be_ref,
             o_ref):
        h2 = h_ref[...] + a_ref[0] + a_ref[1]
        z = jnp.dot(h2, w1_ref[...], preferred_element_type=jnp.float32)
        z = jnp.maximum(z + b1_ref[...], 0.0)
        z = jnp.dot(z, w2_ref[...], preferred_element_type=jnp.float32)
        z = jnp.maximum(z + b2_ref[...], 0.0)
        m = jnp.mean(z, axis=0, keepdims=True)
        c = z - m
        v = jnp.mean(c * c, axis=0, keepdims=True)
        o_ref[...] = c * lax.rsqrt(v + 1e-5) * g_ref[...] + be_ref[...]

    return pl.pallas_call(
        body, out_shape=jax.ShapeDtypeStruct((_N, _D), jnp.float32),
    )(h, agg2, w1, b1.reshape(1, _D), w2, b2.reshape(1, _D),
      g.reshape(1, _D), be.reshape(1, _D))


def _head_tc(h1, h2, h3, b_row, wp1, bp1, wp2, bp2):
    def body(h1_ref, h2_ref, h3_ref, b_ref, wp1_ref, bp1_ref, wp2_ref,
             bp2_ref, y_ref, xc_ref):
        gid = lax.broadcasted_iota(jnp.int32, (_G, _N), 0)
        oht = (b_ref[...] == gid).astype(jnp.float32)
        p1 = jnp.dot(oht, h1_ref[...], preferred_element_type=jnp.float32)
        p2 = jnp.dot(oht, h2_ref[...], preferred_element_type=jnp.float32)
        p3 = jnp.dot(oht, h3_ref[...], preferred_element_type=jnp.float32)
        xc = jnp.concatenate([p1, p2, p3], axis=1)
        y = jnp.dot(xc, wp1_ref[...], preferred_element_type=jnp.float32)
        y = jnp.maximum(y + bp1_ref[...], 0.0)
        y = jnp.dot(y, wp2_ref[...], preferred_element_type=jnp.float32)
        y = y + bp2_ref[...]
        yn = jnp.sqrt(jnp.sum(y * y, axis=1, keepdims=True))
        y_ref[...] = y / jnp.maximum(yn, 1e-12)
        xn = jnp.sqrt(jnp.sum(xc * xc, axis=1, keepdims=True))
        xc_ref[...] = xc / jnp.maximum(xn, 1e-12)

    return pl.pallas_call(
        body,
        out_shape=(jax.ShapeDtypeStruct((_G, 3 * _D), jnp.float32),
                   jax.ShapeDtypeStruct((_G, 3 * _D), jnp.float32)),
    )(h1, h2, h3, b_row, wp1, bp1.reshape(1, 3 * _D), wp2,
      bp2.reshape(1, 3 * _D))


def kernel(x, edge_index, batch,
           W1_0, b1_0, W2_0, b2_0, g_0, be_0,
           W1_1, b1_1, W2_1, b2_1, g_1, be_1,
           W1_2, b1_2, W2_2, b2_2, g_2, be_2,
           Wp1, bp1, Wp2, bp2):
    npad = _EPT - _E // _NW
    src = edge_index[0].astype(jnp.int32).reshape(_NW, _E // _NW)
    dst = edge_index[1].astype(jnp.int32).reshape(_NW, _E // _NW)
    if npad:
        trash = _N + jnp.arange(_NW, dtype=jnp.int32)
        src = jnp.concatenate(
            [src, jnp.zeros((_NW, npad), jnp.int32)], axis=1)
        dst = jnp.concatenate(
            [dst, jnp.broadcast_to(trash[:, None], (_NW, npad))], axis=1)
    b_row = batch.astype(jnp.int32).reshape(1, _N)

    params = [
        (W1_0, b1_0, W2_0, b2_0, g_0, be_0),
        (W1_1, b1_1, W2_1, b2_1, g_1, be_1),
        (W1_2, b1_2, W2_2, b2_2, g_2, be_2),
    ]
    h = x
    hs = []
    for (w1, b1, w2, b2, g, be) in params:
        agg2 = _seg_sum_sc(h, src, dst)
        h = _layer_tc(h, agg2, w1, b1, w2, b2, g, be)
        hs.append(h)
    return _head_tc(hs[0], hs[1], hs[2], b_row, Wp1, bp1, Wp2, bp2)


# speedup vs baseline: 1.0257x; 1.0257x over previous
import functools

import jax
import jax.numpy as jnp
from jax import lax
from jax.experimental import pallas as pl
from jax.experimental.pallas import tpu as pltpu
from jax.experimental.pallas import tpu_sc as plsc

_N = 10000
_E = 320000
_D = 128
_G = 128
_NC = 2
_NS = 16
_NW = _NC * _NS
_K = 80
_EPT = 10000
_NCH = _EPT // _K
_NB = 3
_ZR = 632
_ZL = _N - 15 * _ZR


def _seg_sum_sc(h, src, dst):
    mesh = plsc.VectorSubcoreMesh(
        core_axis_name="c", subcore_axis_name="s",
        num_cores=_NC, num_subcores=_NS)

    @functools.partial(
        pl.kernel, mesh=mesh,
        out_type=jax.ShapeDtypeStruct((_NC, _N, _D), jnp.float32),
        scratch_types=[
            pltpu.VMEM((_EPT,), jnp.int32),
            pltpu.VMEM((_EPT,), jnp.int32),
            pltpu.VMEM((_K,), jnp.int32),
            [pltpu.VMEM((_K, _D), jnp.float32) for _ in range(_NB)],
            pltpu.VMEM_SHARED((_N, _D), jnp.float32),
            pltpu.SemaphoreType.DMA,
            pltpu.SemaphoreType.DMA,
        ],
    )
    def k(h_hbm, src_hbm, dst_hbm, out_hbm, src_v, dst_v, ibuf, rows,
          acc_sh, sem_g, sem_s):
        cid = lax.axis_index("c")
        sid = lax.axis_index("s")
        wid = sid * _NC + cid
        base = pl.multiple_of(sid * _ZR, 8)

        def zrow(i, _):
            def zcol(j, _):
                rows[0][i, pl.ds(j * 16, 16)] = jnp.zeros((16,), jnp.float32)
                return 0
            return lax.fori_loop(0, _D // 16, zcol, 0)
        lax.fori_loop(0, _K, zrow, 0)

        def zacc(nz, r):
            for i in range(nz):
                pltpu.async_copy(
                    rows[0], acc_sh.at[pl.ds(base + i * _K, _K)], sem_s)
            pltpu.async_copy(rows[0].at[pl.ds(0, r)],
                             acc_sh.at[pl.ds(base + nz * _K, r)], sem_s)
            for i in range(nz):
                pltpu.make_async_copy(
                    rows[0], acc_sh.at[pl.ds(0, _K)], sem_s).wait()
            pltpu.make_async_copy(rows[0].at[pl.ds(0, r)],
                                  acc_sh.at[pl.ds(0, r)], sem_s).wait()

        @pl.when(sid < _NS - 1)
        def _():
            zacc(_ZR // _K, _ZR - (_ZR // _K) * _K)

        @pl.when(sid == _NS - 1)
        def _():
            zacc(_ZL // _K, _ZL - (_ZL // _K) * _K)

        pltpu.sync_copy(src_hbm.at[wid], src_v)
        pltpu.sync_copy(dst_hbm.at[wid], dst_v)
        plsc.subcore_barrier()

        def chunk(j):
            return pl.ds(pl.multiple_of(jnp.minimum(j, _NCH - 1) * _K, _K),
                         _K)

        def g_issue(j, b):
            pltpu.async_copy(h_hbm.at[src_v.at[chunk(j)]], rows[b], sem_g)

        def g_wait(b):
            pltpu.make_async_copy(
                h_hbm.at[src_v.at[pl.ds(0, _K)]], rows[b], sem_g).wait()

        def s_issue(b):
            pltpu.async_copy(rows[b], acc_sh.at[ibuf], sem_s, add=True)

        def s_wait(b):
            pltpu.make_async_copy(rows[b], acc_sh.at[ibuf], sem_s).wait()

        def step(j, b):
            off = pl.multiple_of(j * _K, _K)
            for t in range(_K // 16):
                ibuf[pl.ds(t * 16, 16)] = dst_v[pl.ds(off + t * 16, 16)]
            g_wait(b)
            s_issue(b)
            s_wait(b)
            g_issue(j + _NB, b)

        for b in range(_NB):
            g_issue(b, b)

        def body(i, _):
            for b in range(_NB):
                step(i * _NB + b, b)
            return 0
        lax.fori_loop(0, _NCH // _NB, body, 0)

        for j in range(_NCH - _NCH % _NB, _NCH):
            step(j, j % _NB)
        for b in range(_NB):
            g_wait(b)
        plsc.subcore_barrier()

        @pl.when(sid < _NS - 1)
        def _():
            pltpu.sync_copy(acc_sh.at[pl.ds(base, _ZR)],
                            out_hbm.at[cid, pl.ds(base, _ZR)])

        @pl.when(sid == _NS - 1)
        def _():
            pltpu.sync_copy(acc_sh.at[pl.ds(base, _ZL)],
                            out_hbm.at[cid, pl.ds(base, _ZL)])

    return k(h, src, dst)


def _layer_tc(h, agg2, w1, b1, w2, b2, g, be):
    def body(h_ref, a_ref, w1_ref, b1_ref, w2_ref, b2_ref, g_ref, be_ref,
             o_ref):
        h2 = h_ref[...] + a_ref[0] + a_ref[1]
        z = jnp.dot(h2, w1_ref[...], preferred_element_type=jnp.float32)
        z = jnp.maximum(z + b1_ref[...], 0.0)
        z = jnp.dot(z, w2_ref[...], preferred_element_type=jnp.float32)
        z = jnp.maximum(z + b2_ref[...], 0.0)
        m = jnp.mean(z, axis=0, keepdims=True)
        c = z - m
        v = jnp.mean(c * c, axis=0, keepdims=True)
        o_ref[...] = c * lax.rsqrt(v + 1e-5) * g_ref[...] + be_ref[...]

    return pl.pallas_call(
        body, out_shape=jax.ShapeDtypeStruct((_N, _D), jnp.float32),
    )(h, agg2, w1, b1.reshape(1, _D), w2, b2.reshape(1, _D),
      g.reshape(1, _D), be.reshape(1, _D))


def _head_tc(h1, h2, h3, b_row, wp1, bp1, wp2, bp2):
    def body(h1_ref, h2_ref, h3_ref, b_ref, wp1_ref, bp1_ref, wp2_ref,
             bp2_ref, y_ref, xc_ref):
        gid = lax.broadcasted_iota(jnp.int32, (_G, _N), 0)
        oht = (b_ref[...] == gid).astype(jnp.float32)
        p1 = jnp.dot(oht, h1_ref[...], preferred_element_type=jnp.float32)
        p2 = jnp.dot(oht, h2_ref[...], preferred_element_type=jnp.float32)
        p3 = jnp.dot(oht, h3_ref[...], preferred_element_type=jnp.float32)
        xc = jnp.concatenate([p1, p2, p3], axis=1)
        y = jnp.dot(xc, wp1_ref[...], preferred_element_type=jnp.float32)
        y = jnp.maximum(y + bp1_ref[...], 0.0)
        y = jnp.dot(y, wp2_ref[...], preferred_element_type=jnp.float32)
        y = y + bp2_ref[...]
        yn = jnp.sqrt(jnp.sum(y * y, axis=1, keepdims=True))
        y_ref[...] = y / jnp.maximum(yn, 1e-12)
        xn = jnp.sqrt(jnp.sum(xc * xc, axis=1, keepdims=True))
        xc_ref[...] = xc / jnp.maximum(xn, 1e-12)

    return pl.pallas_call(
        body,
        out_shape=(jax.ShapeDtypeStruct((_G, 3 * _D), jnp.float32),
                   jax.ShapeDtypeStruct((_G, 3 * _D), jnp.float32)),
    )(h1, h2, h3, b_row, wp1, bp1.reshape(1, 3 * _D), wp2,
      bp2.reshape(1, 3 * _D))


def kernel(x, edge_index, batch,
           W1_0, b1_0, W2_0, b2_0, g_0, be_0,
           W1_1, b1_1, W2_1, b2_1, g_1, be_1,
           W1_2, b1_2, W2_2, b2_2, g_2, be_2,
           Wp1, bp1, Wp2, bp2):
    npad = _EPT - _E // _NW
    src = edge_index[0].astype(jnp.int32).reshape(_NW, _E // _NW)
    dst = edge_index[1].astype(jnp.int32).reshape(_NW, _E // _NW)
    if npad:
        trash = _N + jnp.arange(_NW, dtype=jnp.int32)
        src = jnp.concatenate(
            [src, jnp.zeros((_NW, npad), jnp.int32)], axis=1)
        dst = jnp.concatenate(
            [dst, jnp.broadcast_to(trash[:, None], (_NW, npad))], axis=1)
    b_row = batch.astype(jnp.int32).reshape(1, _N)

    params = [
        (W1_0, b1_0, W2_0, b2_0, g_0, be_0),
        (W1_1, b1_1, W2_1, b2_1, g_1, be_1),
        (W1_2, b1_2, W2_2, b2_2, g_2, be_2),
    ]
    h = x
    hs = []
    for (w1, b1, w2, b2, g, be) in params:
        agg2 = _seg_sum_sc(h, src, dst)
        h = _layer_tc(h, agg2, w1, b1, w2, b2, g, be)
        hs.append(h)
    return _head_tc(hs[0], hs[1], hs[2], b_row, Wp1, bp1, Wp2, bp2)
